# Initial kernel scaffold; baseline (speedup 1.0000x reference)
import jax
import jax.numpy as jnp
from jax.experimental import pallas as pl


def kernel(x, edge_index, batch, Wl1, bl1, Wr1, Wl2, bl2, Wr2, W3, b3, W4, b4):
    raise NotImplementedError("write your pallas kernel here")


# speedup vs baseline: 5.2083x; 5.2083x over previous
import jax
import jax.numpy as jnp
from jax import lax
from jax.experimental import pallas as pl
from jax.experimental.pallas import tpu as pltpu
from jax.experimental.pallas import tpu_sc as plsc

F32 = jnp.float32
I32 = jnp.int32

_N = 10000
_E = 160000
_G = 128
_NP = 10240
_ROWS_PER_TILE = _NP // 16
_PAD_IDX = _N + 16

_NS = 16
_NC = 2
_K = 128

_CHB = 8
_EPT1 = _E // _NS
_CH1 = -(-_EPT1 // (_K * _CHB)) * _CHB
_EPT1_PAD = _CH1 * _K
_EPG2 = _E // (_NC * _NS)
_CH2 = -(-_EPG2 // (_K * _CHB)) * _CHB
_EPG2_PAD = _CH2 * _K

_NB = _NP // 256


def _dotT(a, w):
    return lax.dot_general(a, w, (((1,), (1,)), ((), ())),
                           preferred_element_type=F32)


def _mm1_body(x_ref, wla_ref, wlb_ref, wr_ref, bl_ref,
              ya_ref, yb_ref, z1_ref):
    xb = x_ref[...]
    ya_ref[...] = _dotT(xb, wla_ref[...])
    yb_ref[...] = _dotT(xb, wlb_ref[...])
    z1_ref[...] = _dotT(xb, wr_ref[...]) + bl_ref[...]


def _run_mm1(xp, wla, wlb, wr1, bl1r):
    return pl.pallas_call(
        _mm1_body,
        grid=(_NB,),
        in_specs=[
            pl.BlockSpec((256, 768), lambda i: (i, 0)),
            pl.BlockSpec((128, 768), lambda i: (0, 0)),
            pl.BlockSpec((128, 768), lambda i: (0, 0)),
            pl.BlockSpec((256, 768), lambda i: (0, 0)),
            pl.BlockSpec((1, 256), lambda i: (0, 0)),
        ],
        out_specs=[
            pl.BlockSpec((256, 128), lambda i: (i, 0)),
            pl.BlockSpec((256, 128), lambda i: (i, 0)),
            pl.BlockSpec((256, 256), lambda i: (i, 0)),
        ],
        out_shape=[
            jax.ShapeDtypeStruct((_NP, 128), F32),
            jax.ShapeDtypeStruct((_NP, 128), F32),
            jax.ShapeDtypeStruct((_NP, 256), F32),
        ],
    )(xp, wla, wlb, wr1, bl1r)


def _mm2_body(aa_ref, ab_ref, z1_ref, d0_ref, d1_ref, wl_ref, wr_ref, bl_ref,
              y2_ref, z2_ref):
    dg = jnp.maximum(d0_ref[...][:, 0:1] + d1_ref[...][:, 0:1], 1.0)
    agg = jnp.concatenate([aa_ref[...], ab_ref[...]], axis=1)
    h1 = jnp.maximum(agg / dg + z1_ref[...], 0.0)
    y2_ref[...] = _dotT(h1, wl_ref[...])
    z2_ref[...] = _dotT(h1, wr_ref[...]) + bl_ref[...]


def _run_mm2(agg, z1, degw, wl2, wr2, bl2r):
    return pl.pallas_call(
        _mm2_body,
        grid=(_NB,),
        in_specs=[
            pl.BlockSpec((256, 128), lambda i: (i, 0)),
            pl.BlockSpec((256, 128), lambda i: (i + _NB, 0)),
            pl.BlockSpec((256, 256), lambda i: (i, 0)),
            pl.BlockSpec((256, 128), lambda i: (i, 0)),
            pl.BlockSpec((256, 128), lambda i: (i + _NB, 0)),
            pl.BlockSpec((128, 256), lambda i: (0, 0)),
            pl.BlockSpec((128, 256), lambda i: (0, 0)),
            pl.BlockSpec((1, 128), lambda i: (0, 0)),
        ],
        out_specs=[
            pl.BlockSpec((256, 128), lambda i: (i, 0)),
            pl.BlockSpec((256, 128), lambda i: (i, 0)),
        ],
        out_shape=[
            jax.ShapeDtypeStruct((_NP, 128), F32),
            jax.ShapeDtypeStruct((_NP, 128), F32),
        ],
    )(agg, agg, z1, degw, degw, wl2, wr2, bl2r)


def _head_body(p0_ref, p1_ref, z2_ref, d0_ref, d1_ref, bat_ref,
               w3_ref, b3_ref, w4_ref, b4_ref,
               out_ref, acc_ref, cnt_ref):
    i = pl.program_id(0)
    nb = pl.num_programs(0)

    @pl.when(i == 0)
    def _():
        acc_ref[...] = jnp.zeros_like(acc_ref)
        cnt_ref[...] = jnp.zeros_like(cnt_ref)

    dg = jnp.maximum(d0_ref[...][:, 0:1] + d1_ref[...][:, 0:1], 1.0)
    h2 = jnp.maximum((p0_ref[...] + p1_ref[...]) / dg + z2_ref[...], 0.0)
    b = bat_ref[0, 0, :]
    onehot = (b[:, None] == lax.broadcasted_iota(I32, (256, _G), 1)).astype(F32)
    acc_ref[...] += lax.dot_general(onehot, h2, (((0,), (0,)), ((), ())),
                                    preferred_element_type=F32)
    cnt_ref[...] += lax.dot_general(onehot, jnp.ones((256, 8), F32),
                                    (((0,), (0,)), ((), ())),
                                    preferred_element_type=F32)

    @pl.when(i == nb - 1)
    def _():
        pool = acc_ref[...] / jnp.maximum(cnt_ref[...][:, 0:1], 1.0)
        t3 = _dotT(pool, w3_ref[...]) + b3_ref[...]
        t4 = _dotT(t3, w4_ref[...]) + b4_ref[...]
        m = jnp.max(t4, axis=-1, keepdims=True)
        e = jnp.exp(t4 - m)
        out_ref[...] = e / jnp.sum(e, axis=-1, keepdims=True)


def _run_head(parts, z2, degw, bat3, w3, b3r, w4, b4r):
    return pl.pallas_call(
        _head_body,
        grid=(_NB,),
        in_specs=[
            pl.BlockSpec((256, 128), lambda i: (i, 0)),
            pl.BlockSpec((256, 128), lambda i: (i + _NB, 0)),
            pl.BlockSpec((256, 128), lambda i: (i, 0)),
            pl.BlockSpec((256, 128), lambda i: (i, 0)),
            pl.BlockSpec((256, 128), lambda i: (i + _NB, 0)),
            pl.BlockSpec((1, 1, 256), lambda i: (i, 0, 0)),
            pl.BlockSpec((64, 128), lambda i: (0, 0)),
            pl.BlockSpec((1, 64), lambda i: (0, 0)),
            pl.BlockSpec((2, 64), lambda i: (0, 0)),
            pl.BlockSpec((1, 2), lambda i: (0, 0)),
        ],
        out_specs=pl.BlockSpec((_G, 2), lambda i: (0, 0)),
        out_shape=jax.ShapeDtypeStruct((_G, 2), F32),
        scratch_shapes=[
            pltpu.VMEM((_G, 128), F32),
            pltpu.VMEM((_G, 8), F32),
        ],
    )(parts, parts, z2, degw, degw, bat3, w3, b3r, w4, b4r)


_MESH = plsc.VectorSubcoreMesh(core_axis_name="c", subcore_axis_name="s")


def _deg_body(dst_hbm, zf_hbm, onesf_hbm,
              degw_hbm,
              dst_v, ones_v, table, sem):
    c = lax.axis_index("c")
    s = lax.axis_index("s")
    f = c * _NS + s
    base = s * _ROWS_PER_TILE

    pltpu.sync_copy(zf_hbm, table.at[pl.ds(base, _ROWS_PER_TILE)])
    pltpu.sync_copy(onesf_hbm, ones_v)

    plsc.subcore_barrier()

    def outer(g, carry):
        pltpu.sync_copy(dst_hbm.at[pl.ds(f * _CH2 + g * _CHB, _CHB)], dst_v)

        def inner(j, carry2):
            pltpu.sync_copy(ones_v, table.at[dst_v.at[j]], add=True)
            return carry2

        return lax.fori_loop(0, _CHB, inner, carry)

    lax.fori_loop(0, _CH2 // _CHB, outer, 0)

    plsc.subcore_barrier()

    pltpu.sync_copy(table.at[pl.ds(base, _ROWS_PER_TILE)],
                    degw_hbm.at[pl.ds(c * _NP + base, _ROWS_PER_TILE)])


def _run_deg(dst2, zf, onesf):
    return pl.kernel(
        _deg_body,
        out_type=[jax.ShapeDtypeStruct((2 * _NP, 128), F32)],
        mesh=_MESH,
        scratch_types=[
            pltpu.VMEM((_CHB, _K), I32),
            pltpu.VMEM((_K, 128), F32),
            pltpu.VMEM_SHARED((_NP, 128), F32),
            pltpu.SemaphoreType.DMA,
        ],
    )(dst2, zf, onesf)


def _agg1_body(yall_hbm, src_hbm, dst_hbm, zf_hbm,
               agg_hbm,
               src_v, dst_v, buf_v, table, sem):
    c = lax.axis_index("c")
    s = lax.axis_index("s")
    base = s * _ROWS_PER_TILE

    pltpu.sync_copy(zf_hbm, table.at[pl.ds(base, _ROWS_PER_TILE)])

    plsc.subcore_barrier()

    idx0 = c * (_NS * _CH1) + s * _CH1

    def outer(g, carry):
        pltpu.sync_copy(src_hbm.at[pl.ds(idx0 + g * _CHB, _CHB)], src_v)
        pltpu.sync_copy(dst_hbm.at[pl.ds(s * _CH1 + g * _CHB, _CHB)], dst_v)

        def inner(j, carry2):
            pltpu.async_copy(yall_hbm.at[src_v.at[j]], buf_v, sem).wait()
            pltpu.sync_copy(buf_v, table.at[dst_v.at[j]], add=True)
            return carry2

        return lax.fori_loop(0, _CHB, inner, carry)

    lax.fori_loop(0, _CH1 // _CHB, outer, 0)

    plsc.subcore_barrier()

    pltpu.sync_copy(table.at[pl.ds(base, _ROWS_PER_TILE)],
                    agg_hbm.at[pl.ds(c * _NP + base, _ROWS_PER_TILE)])


def _run_agg1(yall, src1s, dst1, zf):
    return pl.kernel(
        _agg1_body,
        out_type=[jax.ShapeDtypeStruct((2 * _NP, 128), F32)],
        mesh=_MESH,
        scratch_types=[
            pltpu.VMEM((_CHB, _K), I32),
            pltpu.VMEM((_CHB, _K), I32),
            pltpu.VMEM((_K, 128), F32),
            pltpu.VMEM_SHARED((_NP, 128), F32),
            pltpu.SemaphoreType.DMA,
        ],
    )(yall, src1s, dst1, zf)


def _agg2_body(y2_hbm, src_hbm, dst_hbm, zf_hbm,
               parts_hbm,
               src_v, dst_v, buf_v, table, sem):
    c = lax.axis_index("c")
    s = lax.axis_index("s")
    f = c * _NS + s
    base = s * _ROWS_PER_TILE

    pltpu.sync_copy(zf_hbm, table.at[pl.ds(base, _ROWS_PER_TILE)])

    plsc.subcore_barrier()

    def outer(g, carry):
        blk = pl.ds(f * _CH2 + g * _CHB, _CHB)
        pltpu.sync_copy(src_hbm.at[blk], src_v)
        pltpu.sync_copy(dst_hbm.at[blk], dst_v)

        def inner(j, carry2):
            pltpu.async_copy(y2_hbm.at[src_v.at[j]], buf_v, sem).wait()
            pltpu.sync_copy(buf_v, table.at[dst_v.at[j]], add=True)
            return carry2

        return lax.fori_loop(0, _CHB, inner, carry)

    lax.fori_loop(0, _CH2 // _CHB, outer, 0)

    plsc.subcore_barrier()

    pltpu.sync_copy(table.at[pl.ds(base, _ROWS_PER_TILE)],
                    parts_hbm.at[pl.ds(c * _NP + base, _ROWS_PER_TILE)])


def _run_agg2(y2, src2, dst2, zf):
    return pl.kernel(
        _agg2_body,
        out_type=[jax.ShapeDtypeStruct((2 * _NP, 128), F32)],
        mesh=_MESH,
        scratch_types=[
            pltpu.VMEM((_CHB, _K), I32),
            pltpu.VMEM((_CHB, _K), I32),
            pltpu.VMEM((_K, 128), F32),
            pltpu.VMEM_SHARED((_NP, 128), F32),
            pltpu.SemaphoreType.DMA,
        ],
    )(y2, src2, dst2, zf)


def _edge_layout(v, groups, per_group_pad, chunks):
    v = v.reshape(groups, -1)
    pad = per_group_pad - v.shape[1]
    v = jnp.concatenate(
        [v, jnp.full((groups, pad), _PAD_IDX, dtype=I32)], axis=1)
    return v.reshape(groups * chunks, _K)


@jax.jit
def kernel(x, edge_index, batch, Wl1, bl1, Wr1, Wl2, bl2, Wr2, W3, b3, W4, b4):
    src = edge_index[0]
    dst = edge_index[1]

    xp = jnp.pad(x, ((0, _NP - _N), (0, 0)))
    src1 = _edge_layout(src, _NS, _EPT1_PAD, _CH1)
    src1s = jnp.concatenate([src1, src1 + _NP], axis=0)
    dst1 = _edge_layout(dst, _NS, _EPT1_PAD, _CH1)
    src2 = _edge_layout(src, _NC * _NS, _EPG2_PAD, _CH2)
    dst2 = _edge_layout(dst, _NC * _NS, _EPG2_PAD, _CH2)
    batp = jnp.concatenate(
        [batch, jnp.full((_NP - _N,), _G, dtype=batch.dtype)]
    ).reshape(_NB, 1, 256)

    zf = jnp.zeros((_ROWS_PER_TILE, 128), F32)
    onesf = jnp.ones((_K, 128), F32)

    degw = _run_deg(dst2, zf, onesf)[0]
    ya, yb, z1 = _run_mm1(xp, Wl1[:128], Wl1[128:], Wr1, bl1.reshape(1, 256))
    yall = jnp.concatenate([ya, yb], axis=0)

    agg = _run_agg1(yall, src1s, dst1, zf)[0]
    y2, z2 = _run_mm2(agg, z1, degw, Wl2, Wr2, bl2.reshape(1, 128))
    parts = _run_agg2(y2, src2, dst2, zf)[0]
    out = _run_head(parts, z2, degw, batp, W3, b3.reshape(1, 64),
                    W4, b4.reshape(1, 2))
    return out


# speedup vs baseline: 5.8859x; 1.1301x over previous
import jax
import jax.numpy as jnp
from jax import lax
from jax.experimental import pallas as pl
from jax.experimental.pallas import tpu as pltpu
from jax.experimental.pallas import tpu_sc as plsc

F32 = jnp.float32
I32 = jnp.int32

_N = 10000
_E = 160000
_G = 128
_NP = 10240
_ROWS_PER_TILE = _NP // 16
_PAD_IDX = _N + 16

_NS = 16
_NC = 2
_K = 128

_CHB1 = 16
_EPT1 = _E // _NS
_CH1 = -(-_EPT1 // (_K * _CHB1)) * _CHB1
_EPT1_PAD = _CH1 * _K
_CHB2 = 8
_EPG2 = _E // (_NC * _NS)
_CH2 = -(-_EPG2 // (_K * _CHB2)) * _CHB2
_EPG2_PAD = _CH2 * _K

_NB = _NP // 256


def _dotT(a, w):
    return lax.dot_general(a, w, (((1,), (1,)), ((), ())),
                           preferred_element_type=F32)


def _mm1_body(x_ref, wla_ref, wlb_ref, wr_ref, bl_ref,
              ya_ref, yb_ref, z1_ref):
    xb = x_ref[...]
    ya_ref[...] = _dotT(xb, wla_ref[...])
    yb_ref[...] = _dotT(xb, wlb_ref[...])
    z1_ref[...] = _dotT(xb, wr_ref[...]) + bl_ref[...]


def _run_mm1(xp, wla, wlb, wr1, bl1r):
    return pl.pallas_call(
        _mm1_body,
        grid=(_NB,),
        in_specs=[
            pl.BlockSpec((256, 768), lambda i: (i, 0)),
            pl.BlockSpec((128, 768), lambda i: (0, 0)),
            pl.BlockSpec((128, 768), lambda i: (0, 0)),
            pl.BlockSpec((256, 768), lambda i: (0, 0)),
            pl.BlockSpec((1, 256), lambda i: (0, 0)),
        ],
        out_specs=[
            pl.BlockSpec((256, 128), lambda i: (i, 0)),
            pl.BlockSpec((256, 128), lambda i: (i, 0)),
            pl.BlockSpec((256, 256), lambda i: (i, 0)),
        ],
        out_shape=[
            jax.ShapeDtypeStruct((_NP, 128), F32),
            jax.ShapeDtypeStruct((_NP, 128), F32),
            jax.ShapeDtypeStruct((_NP, 256), F32),
        ],
    )(xp, wla, wlb, wr1, bl1r)


def _mm2_body(aa_ref, ab_ref, z1_ref, d0_ref, d1_ref, wl_ref, wr_ref, bl_ref,
              y2_ref, z2_ref):
    dg = jnp.maximum(d0_ref[...][:, 0:1] + d1_ref[...][:, 0:1], 1.0)
    agg = jnp.concatenate([aa_ref[...], ab_ref[...]], axis=1)
    h1 = jnp.maximum(agg / dg + z1_ref[...], 0.0)
    y2_ref[...] = _dotT(h1, wl_ref[...])
    z2_ref[...] = _dotT(h1, wr_ref[...]) + bl_ref[...]


def _run_mm2(agg, z1, degw, wl2, wr2, bl2r):
    return pl.pallas_call(
        _mm2_body,
        grid=(_NB,),
        in_specs=[
            pl.BlockSpec((256, 128), lambda i: (i, 0)),
            pl.BlockSpec((256, 128), lambda i: (i + _NB, 0)),
            pl.BlockSpec((256, 256), lambda i: (i, 0)),
            pl.BlockSpec((256, 128), lambda i: (i, 0)),
            pl.BlockSpec((256, 128), lambda i: (i + _NB, 0)),
            pl.BlockSpec((128, 256), lambda i: (0, 0)),
            pl.BlockSpec((128, 256), lambda i: (0, 0)),
            pl.BlockSpec((1, 128), lambda i: (0, 0)),
        ],
        out_specs=[
            pl.BlockSpec((256, 128), lambda i: (i, 0)),
            pl.BlockSpec((256, 128), lambda i: (i, 0)),
        ],
        out_shape=[
            jax.ShapeDtypeStruct((_NP, 128), F32),
            jax.ShapeDtypeStruct((_NP, 128), F32),
        ],
    )(agg, agg, z1, degw, degw, wl2, wr2, bl2r)


def _head_body(p0_ref, p1_ref, z2_ref, d0_ref, d1_ref, bat_ref,
               w3_ref, b3_ref, w4_ref, b4_ref,
               out_ref, acc_ref, cnt_ref):
    i = pl.program_id(0)
    nb = pl.num_programs(0)

    @pl.when(i == 0)
    def _():
        acc_ref[...] = jnp.zeros_like(acc_ref)
        cnt_ref[...] = jnp.zeros_like(cnt_ref)

    dg = jnp.maximum(d0_ref[...][:, 0:1] + d1_ref[...][:, 0:1], 1.0)
    h2 = jnp.maximum((p0_ref[...] + p1_ref[...]) / dg + z2_ref[...], 0.0)
    b = bat_ref[0, 0, :]
    onehot = (b[:, None] == lax.broadcasted_iota(I32, (256, _G), 1)).astype(F32)
    acc_ref[...] += lax.dot_general(onehot, h2, (((0,), (0,)), ((), ())),
                                    preferred_element_type=F32)
    cnt_ref[...] += lax.dot_general(onehot, jnp.ones((256, 8), F32),
                                    (((0,), (0,)), ((), ())),
                                    preferred_element_type=F32)

    @pl.when(i == nb - 1)
    def _():
        pool = acc_ref[...] / jnp.maximum(cnt_ref[...][:, 0:1], 1.0)
        t3 = _dotT(pool, w3_ref[...]) + b3_ref[...]
        t4 = _dotT(t3, w4_ref[...]) + b4_ref[...]
        m = jnp.max(t4, axis=-1, keepdims=True)
        e = jnp.exp(t4 - m)
        out_ref[...] = e / jnp.sum(e, axis=-1, keepdims=True)


def _run_head(parts, z2, degw, bat3, w3, b3r, w4, b4r):
    return pl.pallas_call(
        _head_body,
        grid=(_NB,),
        in_specs=[
            pl.BlockSpec((256, 128), lambda i: (i, 0)),
            pl.BlockSpec((256, 128), lambda i: (i + _NB, 0)),
            pl.BlockSpec((256, 128), lambda i: (i, 0)),
            pl.BlockSpec((256, 128), lambda i: (i, 0)),
            pl.BlockSpec((256, 128), lambda i: (i + _NB, 0)),
            pl.BlockSpec((1, 1, 256), lambda i: (i, 0, 0)),
            pl.BlockSpec((64, 128), lambda i: (0, 0)),
            pl.BlockSpec((1, 64), lambda i: (0, 0)),
            pl.BlockSpec((2, 64), lambda i: (0, 0)),
            pl.BlockSpec((1, 2), lambda i: (0, 0)),
        ],
        out_specs=pl.BlockSpec((_G, 2), lambda i: (0, 0)),
        out_shape=jax.ShapeDtypeStruct((_G, 2), F32),
        scratch_shapes=[
            pltpu.VMEM((_G, 128), F32),
            pltpu.VMEM((_G, 8), F32),
        ],
    )(parts, parts, z2, degw, degw, bat3, w3, b3r, w4, b4r)


_MESH = plsc.VectorSubcoreMesh(core_axis_name="c", subcore_axis_name="s")


def _deg_body(dst_hbm, zf_hbm, onesf_hbm,
              degw_hbm,
              dst_v, ones_v, table, sem):
    c = lax.axis_index("c")
    s = lax.axis_index("s")
    f = c * _NS + s
    base = s * _ROWS_PER_TILE

    pltpu.sync_copy(zf_hbm, table.at[pl.ds(base, _ROWS_PER_TILE)])
    pltpu.sync_copy(onesf_hbm, ones_v)

    plsc.subcore_barrier()

    def outer(g, carry):
        pltpu.sync_copy(dst_hbm.at[pl.ds(f * _CH2 + g * _CHB2, _CHB2)], dst_v)
        for j in range(_CHB2):
            pltpu.sync_copy(ones_v, table.at[dst_v.at[j]], add=True)
        return carry

    lax.fori_loop(0, _CH2 // _CHB2, outer, 0)

    plsc.subcore_barrier()

    pltpu.sync_copy(table.at[pl.ds(base, _ROWS_PER_TILE)],
                    degw_hbm.at[pl.ds(c * _NP + base, _ROWS_PER_TILE)])


def _run_deg(dst2, zf, onesf):
    return pl.kernel(
        _deg_body,
        out_type=[jax.ShapeDtypeStruct((2 * _NP, 128), F32)],
        mesh=_MESH,
        scratch_types=[
            pltpu.VMEM((_CHB2, _K), I32),
            pltpu.VMEM((_K, 128), F32),
            pltpu.VMEM_SHARED((_NP, 128), F32),
            pltpu.SemaphoreType.DMA,
        ],
    )(dst2, zf, onesf)


def _pipelined_edges(table_hbm, src_hbm, dst_hbm, src_v, dst_v,
                     buf0, buf1, sem0, sem1, table,
                     idx_src0, idx_dst0, n_refill, chb):
    bufs = (buf0, buf1)
    sems = (sem0, sem1)

    def outer(g, carry):
        pltpu.sync_copy(src_hbm.at[pl.ds(idx_src0 + g * chb, chb)], src_v)
        pltpu.sync_copy(dst_hbm.at[pl.ds(idx_dst0 + g * chb, chb)], dst_v)
        cps = [None, None]
        cps[0] = pltpu.async_copy(table_hbm.at[src_v.at[0]], bufs[0], sems[0])
        for j in range(chb):
            if j + 1 < chb:
                cps[(j + 1) % 2] = pltpu.async_copy(
                    table_hbm.at[src_v.at[j + 1]],
                    bufs[(j + 1) % 2], sems[(j + 1) % 2])
            cps[j % 2].wait()
            pltpu.sync_copy(bufs[j % 2], table.at[dst_v.at[j]], add=True)
        return carry

    lax.fori_loop(0, n_refill, outer, 0)


def _agg1_body(yall_hbm, src_hbm, dst_hbm, zf_hbm,
               agg_hbm,
               src_v, dst_v, buf0, buf1, table, sem0, sem1):
    c = lax.axis_index("c")
    s = lax.axis_index("s")
    base = s * _ROWS_PER_TILE

    pltpu.sync_copy(zf_hbm, table.at[pl.ds(base, _ROWS_PER_TILE)])

    plsc.subcore_barrier()

    _pipelined_edges(yall_hbm, src_hbm, dst_hbm, src_v, dst_v,
                     buf0, buf1, sem0, sem1, table,
                     c * (_NS * _CH1) + s * _CH1, s * _CH1,
                     _CH1 // _CHB1, _CHB1)

    plsc.subcore_barrier()

    pltpu.sync_copy(table.at[pl.ds(base, _ROWS_PER_TILE)],
                    agg_hbm.at[pl.ds(c * _NP + base, _ROWS_PER_TILE)])


def _run_agg1(yall, src1s, dst1, zf):
    return pl.kernel(
        _agg1_body,
        out_type=[jax.ShapeDtypeStruct((2 * _NP, 128), F32)],
        mesh=_MESH,
        scratch_types=[
            pltpu.VMEM((_CHB1, _K), I32),
            pltpu.VMEM((_CHB1, _K), I32),
            pltpu.VMEM((_K, 128), F32),
            pltpu.VMEM((_K, 128), F32),
            pltpu.VMEM_SHARED((_NP, 128), F32),
            pltpu.SemaphoreType.DMA,
            pltpu.SemaphoreType.DMA,
        ],
    )(yall, src1s, dst1, zf)


def _agg2_body(y2_hbm, src_hbm, dst_hbm, zf_hbm,
               parts_hbm,
               src_v, dst_v, buf0, buf1, table, sem0, sem1):
    c = lax.axis_index("c")
    s = lax.axis_index("s")
    f = c * _NS + s
    base = s * _ROWS_PER_TILE

    pltpu.sync_copy(zf_hbm, table.at[pl.ds(base, _ROWS_PER_TILE)])

    plsc.subcore_barrier()

    _pipelined_edges(y2_hbm, src_hbm, dst_hbm, src_v, dst_v,
                     buf0, buf1, sem0, sem1, table,
                     f * _CH2, f * _CH2,
                     _CH2 // _CHB2, _CHB2)

    plsc.subcore_barrier()

    pltpu.sync_copy(table.at[pl.ds(base, _ROWS_PER_TILE)],
                    parts_hbm.at[pl.ds(c * _NP + base, _ROWS_PER_TILE)])


def _run_agg2(y2, src2, dst2, zf):
    return pl.kernel(
        _agg2_body,
        out_type=[jax.ShapeDtypeStruct((2 * _NP, 128), F32)],
        mesh=_MESH,
        scratch_types=[
            pltpu.VMEM((_CHB2, _K), I32),
            pltpu.VMEM((_CHB2, _K), I32),
            pltpu.VMEM((_K, 128), F32),
            pltpu.VMEM((_K, 128), F32),
            pltpu.VMEM_SHARED((_NP, 128), F32),
            pltpu.SemaphoreType.DMA,
            pltpu.SemaphoreType.DMA,
        ],
    )(y2, src2, dst2, zf)


def _edge_layout(v, groups, per_group_pad, chunks):
    v = v.reshape(groups, -1)
    pad = per_group_pad - v.shape[1]
    v = jnp.concatenate(
        [v, jnp.full((groups, pad), _PAD_IDX, dtype=I32)], axis=1)
    return v.reshape(groups * chunks, _K)


@jax.jit
def kernel(x, edge_index, batch, Wl1, bl1, Wr1, Wl2, bl2, Wr2, W3, b3, W4, b4):
    src = edge_index[0]
    dst = edge_index[1]

    xp = jnp.pad(x, ((0, _NP - _N), (0, 0)))
    src1 = _edge_layout(src, _NS, _EPT1_PAD, _CH1)
    src1s = jnp.concatenate([src1, src1 + _NP], axis=0)
    dst1 = _edge_layout(dst, _NS, _EPT1_PAD, _CH1)
    src2 = _edge_layout(src, _NC * _NS, _EPG2_PAD, _CH2)
    dst2 = _edge_layout(dst, _NC * _NS, _EPG2_PAD, _CH2)
    batp = jnp.concatenate(
        [batch, jnp.full((_NP - _N,), _G, dtype=batch.dtype)]
    ).reshape(_NB, 1, 256)

    zf = jnp.zeros((_ROWS_PER_TILE, 128), F32)
    onesf = jnp.ones((_K, 128), F32)

    degw = _run_deg(dst2, zf, onesf)[0]
    ya, yb, z1 = _run_mm1(xp, Wl1[:128], Wl1[128:], Wr1, bl1.reshape(1, 256))
    yall = jnp.concatenate([ya, yb], axis=0)

    agg = _run_agg1(yall, src1s, dst1, zf)[0]
    y2, z2 = _run_mm2(agg, z1, degw, Wl2, Wr2, bl2.reshape(1, 128))
    parts = _run_agg2(y2, src2, dst2, zf)[0]
    out = _run_head(parts, z2, degw, batp, W3, b3.reshape(1, 64),
                    W4, b4.reshape(1, 2))
    return out


# speedup vs baseline: 5.8904x; 1.0008x over previous
import jax
import jax.numpy as jnp
from jax import lax
from jax.experimental import pallas as pl
from jax.experimental.pallas import tpu as pltpu
from jax.experimental.pallas import tpu_sc as plsc

F32 = jnp.float32
I32 = jnp.int32

_N = 10000
_E = 160000
_G = 128
_NP = 10240
_ROWS_PER_TILE = _NP // 16
_PAD_IDX = _N + 16

_NS = 16
_NC = 2
_K = 128

_CHB1 = 16
_EPT1 = _E // _NS
_CH1 = -(-_EPT1 // (_K * _CHB1)) * _CHB1
_EPT1_PAD = _CH1 * _K
_CHB2 = 8
_EPG2 = _E // (_NC * _NS)
_CH2 = -(-_EPG2 // (_K * _CHB2)) * _CHB2
_EPG2_PAD = _CH2 * _K

_NB = _NP // 256


def _dotT(a, w):
    return lax.dot_general(a, w, (((1,), (1,)), ((), ())),
                           preferred_element_type=F32)


def _mm1_body(x_ref, wla_ref, wlb_ref, wr_ref, bl_ref,
              ya_ref, yb_ref, z1_ref):
    xb = x_ref[...]
    ya_ref[...] = _dotT(xb, wla_ref[...])
    yb_ref[...] = _dotT(xb, wlb_ref[...])
    z1_ref[...] = _dotT(xb, wr_ref[...]) + bl_ref[...]


def _run_mm1(xp, wla, wlb, wr1, bl1r):
    return pl.pallas_call(
        _mm1_body,
        grid=(_NB,),
        in_specs=[
            pl.BlockSpec((256, 768), lambda i: (i, 0)),
            pl.BlockSpec((128, 768), lambda i: (0, 0)),
            pl.BlockSpec((128, 768), lambda i: (0, 0)),
            pl.BlockSpec((256, 768), lambda i: (0, 0)),
            pl.BlockSpec((1, 256), lambda i: (0, 0)),
        ],
        out_specs=[
            pl.BlockSpec((256, 128), lambda i: (i, 0)),
            pl.BlockSpec((256, 128), lambda i: (i, 0)),
            pl.BlockSpec((256, 256), lambda i: (i, 0)),
        ],
        out_shape=[
            jax.ShapeDtypeStruct((_NP, 128), F32),
            jax.ShapeDtypeStruct((_NP, 128), F32),
            jax.ShapeDtypeStruct((_NP, 256), F32),
        ],
    )(xp, wla, wlb, wr1, bl1r)


def _mm2_body(aa_ref, ab_ref, z1_ref, d0_ref, d1_ref, wl_ref, wr_ref, bl_ref,
              y2_ref, z2_ref):
    dg = jnp.maximum(d0_ref[...][:, 0:1] + d1_ref[...][:, 0:1], 1.0)
    agg = jnp.concatenate([aa_ref[...], ab_ref[...]], axis=1)
    h1 = jnp.maximum(agg / dg + z1_ref[...], 0.0)
    y2_ref[...] = _dotT(h1, wl_ref[...])
    z2_ref[...] = _dotT(h1, wr_ref[...]) + bl_ref[...]


def _run_mm2(agg, z1, degw, wl2, wr2, bl2r):
    return pl.pallas_call(
        _mm2_body,
        grid=(_NB,),
        in_specs=[
            pl.BlockSpec((256, 128), lambda i: (i, 0)),
            pl.BlockSpec((256, 128), lambda i: (i + _NB, 0)),
            pl.BlockSpec((256, 256), lambda i: (i, 0)),
            pl.BlockSpec((256, 128), lambda i: (i, 0)),
            pl.BlockSpec((256, 128), lambda i: (i + _NB, 0)),
            pl.BlockSpec((128, 256), lambda i: (0, 0)),
            pl.BlockSpec((128, 256), lambda i: (0, 0)),
            pl.BlockSpec((1, 128), lambda i: (0, 0)),
        ],
        out_specs=[
            pl.BlockSpec((256, 128), lambda i: (i, 0)),
            pl.BlockSpec((256, 128), lambda i: (i, 0)),
        ],
        out_shape=[
            jax.ShapeDtypeStruct((_NP, 128), F32),
            jax.ShapeDtypeStruct((_NP, 128), F32),
        ],
    )(agg, agg, z1, degw, degw, wl2, wr2, bl2r)


def _head_body(p0_ref, p1_ref, z2_ref, d0_ref, d1_ref, bat_ref,
               w3_ref, b3_ref, w4_ref, b4_ref,
               out_ref, acc_ref, cnt_ref):
    i = pl.program_id(0)
    nb = pl.num_programs(0)

    @pl.when(i == 0)
    def _():
        acc_ref[...] = jnp.zeros_like(acc_ref)
        cnt_ref[...] = jnp.zeros_like(cnt_ref)

    dg = jnp.maximum(d0_ref[...][:, 0:1] + d1_ref[...][:, 0:1], 1.0)
    h2 = jnp.maximum((p0_ref[...] + p1_ref[...]) / dg + z2_ref[...], 0.0)
    b = bat_ref[0, 0, :]
    onehot = (b[:, None] == lax.broadcasted_iota(I32, (256, _G), 1)).astype(F32)
    acc_ref[...] += lax.dot_general(onehot, h2, (((0,), (0,)), ((), ())),
                                    preferred_element_type=F32)
    cnt_ref[...] += lax.dot_general(onehot, jnp.ones((256, 8), F32),
                                    (((0,), (0,)), ((), ())),
                                    preferred_element_type=F32)

    @pl.when(i == nb - 1)
    def _():
        pool = acc_ref[...] / jnp.maximum(cnt_ref[...][:, 0:1], 1.0)
        t3 = _dotT(pool, w3_ref[...]) + b3_ref[...]
        t4 = _dotT(t3, w4_ref[...]) + b4_ref[...]
        m = jnp.max(t4, axis=-1, keepdims=True)
        e = jnp.exp(t4 - m)
        out_ref[...] = e / jnp.sum(e, axis=-1, keepdims=True)


def _run_head(parts, z2, degw, bat3, w3, b3r, w4, b4r):
    return pl.pallas_call(
        _head_body,
        grid=(_NB,),
        in_specs=[
            pl.BlockSpec((256, 128), lambda i: (i, 0)),
            pl.BlockSpec((256, 128), lambda i: (i + _NB, 0)),
            pl.BlockSpec((256, 128), lambda i: (i, 0)),
            pl.BlockSpec((256, 128), lambda i: (i, 0)),
            pl.BlockSpec((256, 128), lambda i: (i + _NB, 0)),
            pl.BlockSpec((1, 1, 256), lambda i: (i, 0, 0)),
            pl.BlockSpec((64, 128), lambda i: (0, 0)),
            pl.BlockSpec((1, 64), lambda i: (0, 0)),
            pl.BlockSpec((2, 64), lambda i: (0, 0)),
            pl.BlockSpec((1, 2), lambda i: (0, 0)),
        ],
        out_specs=pl.BlockSpec((_G, 2), lambda i: (0, 0)),
        out_shape=jax.ShapeDtypeStruct((_G, 2), F32),
        scratch_shapes=[
            pltpu.VMEM((_G, 128), F32),
            pltpu.VMEM((_G, 8), F32),
        ],
    )(parts, parts, z2, degw, degw, bat3, w3, b3r, w4, b4r)


_MESH = plsc.VectorSubcoreMesh(core_axis_name="c", subcore_axis_name="s")


def _deg_body(dst_hbm, zf_hbm, onesf_hbm,
              degw_hbm,
              dst_v, ones_v, table, sem):
    c = lax.axis_index("c")
    s = lax.axis_index("s")
    f = c * _NS + s
    base = s * _ROWS_PER_TILE

    pltpu.sync_copy(zf_hbm, table.at[pl.ds(base, _ROWS_PER_TILE)])
    pltpu.sync_copy(onesf_hbm, ones_v)

    plsc.subcore_barrier()

    def outer(g, carry):
        pltpu.sync_copy(dst_hbm.at[pl.ds(f * _CH2 + g * _CHB2, _CHB2)], dst_v)
        for j in range(_CHB2):
            pltpu.sync_copy(ones_v, table.at[dst_v.at[j]], add=True)
        return carry

    lax.fori_loop(0, _CH2 // _CHB2, outer, 0)

    plsc.subcore_barrier()

    pltpu.sync_copy(table.at[pl.ds(base, _ROWS_PER_TILE)],
                    degw_hbm.at[pl.ds(c * _NP + base, _ROWS_PER_TILE)])


def _run_deg(dst2, zf, onesf):
    return pl.kernel(
        _deg_body,
        out_type=[jax.ShapeDtypeStruct((2 * _NP, 128), F32)],
        mesh=_MESH,
        scratch_types=[
            pltpu.VMEM((_CHB2, _K), I32),
            pltpu.VMEM((_K, 128), F32),
            pltpu.VMEM_SHARED((_NP, 128), F32),
            pltpu.SemaphoreType.DMA,
        ],
    )(dst2, zf, onesf)


def _pipelined_edges(table_hbm, src_hbm, dst_hbm, src_v, dst_v,
                     buf0, buf1, sem0, sem1, table,
                     idx_src0, idx_dst0, n_refill, chb):
    bufs = (buf0, buf1)
    sems = (sem0, sem1)

    def outer(g, carry):
        pltpu.sync_copy(src_hbm.at[pl.ds(idx_src0 + g * chb, chb)], src_v)
        pltpu.sync_copy(dst_hbm.at[pl.ds(idx_dst0 + g * chb, chb)], dst_v)
        cps = [None, None]
        cps[0] = pltpu.async_copy(table_hbm.at[src_v.at[0]], bufs[0], sems[0])
        cps[1] = pltpu.async_copy(table_hbm.at[src_v.at[1]], bufs[1], sems[1])
        for j in range(chb):
            cps[j % 2].wait()
            pltpu.sync_copy(bufs[j % 2], table.at[dst_v.at[j]], add=True)
            if j + 2 < chb:
                cps[j % 2] = pltpu.async_copy(
                    table_hbm.at[src_v.at[j + 2]],
                    bufs[j % 2], sems[j % 2])
        return carry

    lax.fori_loop(0, n_refill, outer, 0)


def _agg1_body(yall_hbm, src_hbm, dst_hbm, zf_hbm,
               agg_hbm,
               src_v, dst_v, buf0, buf1, table, sem0, sem1):
    c = lax.axis_index("c")
    s = lax.axis_index("s")
    base = s * _ROWS_PER_TILE

    pltpu.sync_copy(zf_hbm, table.at[pl.ds(base, _ROWS_PER_TILE)])

    plsc.subcore_barrier()

    _pipelined_edges(yall_hbm, src_hbm, dst_hbm, src_v, dst_v,
                     buf0, buf1, sem0, sem1, table,
                     c * (_NS * _CH1) + s * _CH1, s * _CH1,
                     _CH1 // _CHB1, _CHB1)

    plsc.subcore_barrier()

    pltpu.sync_copy(table.at[pl.ds(base, _ROWS_PER_TILE)],
                    agg_hbm.at[pl.ds(c * _NP + base, _ROWS_PER_TILE)])


def _run_agg1(yall, src1s, dst1, zf):
    return pl.kernel(
        _agg1_body,
        out_type=[jax.ShapeDtypeStruct((2 * _NP, 128), F32)],
        mesh=_MESH,
        scratch_types=[
            pltpu.VMEM((_CHB1, _K), I32),
            pltpu.VMEM((_CHB1, _K), I32),
            pltpu.VMEM((_K, 128), F32),
            pltpu.VMEM((_K, 128), F32),
            pltpu.VMEM_SHARED((_NP, 128), F32),
            pltpu.SemaphoreType.DMA,
            pltpu.SemaphoreType.DMA,
        ],
    )(yall, src1s, dst1, zf)


def _agg2_body(y2_hbm, src_hbm, dst_hbm, zf_hbm,
               parts_hbm,
               src_v, dst_v, buf0, buf1, table, sem0, sem1):
    c = lax.axis_index("c")
    s = lax.axis_index("s")
    f = c * _NS + s
    base = s * _ROWS_PER_TILE

    pltpu.sync_copy(zf_hbm, table.at[pl.ds(base, _ROWS_PER_TILE)])

    plsc.subcore_barrier()

    _pipelined_edges(y2_hbm, src_hbm, dst_hbm, src_v, dst_v,
                     buf0, buf1, sem0, sem1, table,
                     f * _CH2, f * _CH2,
                     _CH2 // _CHB2, _CHB2)

    plsc.subcore_barrier()

    pltpu.sync_copy(table.at[pl.ds(base, _ROWS_PER_TILE)],
                    parts_hbm.at[pl.ds(c * _NP + base, _ROWS_PER_TILE)])


def _run_agg2(y2, src2, dst2, zf):
    return pl.kernel(
        _agg2_body,
        out_type=[jax.ShapeDtypeStruct((2 * _NP, 128), F32)],
        mesh=_MESH,
        scratch_types=[
            pltpu.VMEM((_CHB2, _K), I32),
            pltpu.VMEM((_CHB2, _K), I32),
            pltpu.VMEM((_K, 128), F32),
            pltpu.VMEM((_K, 128), F32),
            pltpu.VMEM_SHARED((_NP, 128), F32),
            pltpu.SemaphoreType.DMA,
            pltpu.SemaphoreType.DMA,
        ],
    )(y2, src2, dst2, zf)


def _edge_layout(v, groups, per_group_pad, chunks):
    v = v.reshape(groups, -1)
    pad = per_group_pad - v.shape[1]
    v = jnp.concatenate(
        [v, jnp.full((groups, pad), _PAD_IDX, dtype=I32)], axis=1)
    return v.reshape(groups * chunks, _K)


@jax.jit
def kernel(x, edge_index, batch, Wl1, bl1, Wr1, Wl2, bl2, Wr2, W3, b3, W4, b4):
    src = edge_index[0]
    dst = edge_index[1]

    xp = jnp.pad(x, ((0, _NP - _N), (0, 0)))
    src1 = _edge_layout(src, _NS, _EPT1_PAD, _CH1)
    src1s = jnp.concatenate([src1, src1 + _NP], axis=0)
    dst1 = _edge_layout(dst, _NS, _EPT1_PAD, _CH1)
    src2 = _edge_layout(src, _NC * _NS, _EPG2_PAD, _CH2)
    dst2 = _edge_layout(dst, _NC * _NS, _EPG2_PAD, _CH2)
    batp = jnp.concatenate(
        [batch, jnp.full((_NP - _N,), _G, dtype=batch.dtype)]
    ).reshape(_NB, 1, 256)

    zf = jnp.zeros((_ROWS_PER_TILE, 128), F32)
    onesf = jnp.ones((_K, 128), F32)

    degw = _run_deg(dst2, zf, onesf)[0]
    ya, yb, z1 = _run_mm1(xp, Wl1[:128], Wl1[128:], Wr1, bl1.reshape(1, 256))
    yall = jnp.concatenate([ya, yb], axis=0)

    agg = _run_agg1(yall, src1s, dst1, zf)[0]
    y2, z2 = _run_mm2(agg, z1, degw, Wl2, Wr2, bl2.reshape(1, 128))
    parts = _run_agg2(y2, src2, dst2, zf)[0]
    out = _run_head(parts, z2, degw, batp, W3, b3.reshape(1, 64),
                    W4, b4.reshape(1, 2))
    return out


# speedup vs baseline: 6.4714x; 1.0986x over previous
import jax
import jax.numpy as jnp
from jax import lax
from jax.experimental import pallas as pl
from jax.experimental.pallas import tpu as pltpu
from jax.experimental.pallas import tpu_sc as plsc

F32 = jnp.float32
BF16 = jnp.bfloat16
I32 = jnp.int32

_N = 10000
_E = 160000
_G = 128
_NP = 10240
_ROWS_PER_TILE = _NP // 16
_PAD_IDX = _N + 16

_NS = 16
_NC = 2
_K = 128

_CHB1 = 16
_EPT1 = _E // _NS
_CH1 = -(-_EPT1 // (_K * _CHB1)) * _CHB1
_EPT1_PAD = _CH1 * _K
_CHB2 = 8
_EPG2 = _E // (_NC * _NS)
_CH2 = -(-_EPG2 // (_K * _CHB2)) * _CHB2
_EPG2_PAD = _CH2 * _K

_NB = _NP // 256


def _dotT(a, w):
    return lax.dot_general(a, w, (((1,), (1,)), ((), ())),
                           preferred_element_type=F32)


def _mm1_body(x_ref, wla_ref, wlb_ref, wr_ref, bl_ref,
              ya_ref, yb_ref, z1_ref):
    xb = x_ref[...]
    ya_ref[...] = _dotT(xb, wla_ref[...])
    yb_ref[...] = _dotT(xb, wlb_ref[...])
    z1_ref[...] = _dotT(xb, wr_ref[...]) + bl_ref[...]


def _run_mm1(xp, wla, wlb, wr1, bl1r):
    return pl.pallas_call(
        _mm1_body,
        grid=(_NB,),
        in_specs=[
            pl.BlockSpec((256, 768), lambda i: (i, 0)),
            pl.BlockSpec((128, 768), lambda i: (0, 0)),
            pl.BlockSpec((128, 768), lambda i: (0, 0)),
            pl.BlockSpec((256, 768), lambda i: (0, 0)),
            pl.BlockSpec((1, 256), lambda i: (0, 0)),
        ],
        out_specs=[
            pl.BlockSpec((256, 128), lambda i: (i, 0)),
            pl.BlockSpec((256, 128), lambda i: (i, 0)),
            pl.BlockSpec((256, 256), lambda i: (i, 0)),
        ],
        out_shape=[
            jax.ShapeDtypeStruct((_NP, 128), F32),
            jax.ShapeDtypeStruct((_NP, 128), F32),
            jax.ShapeDtypeStruct((_NP, 256), F32),
        ],
    )(xp, wla, wlb, wr1, bl1r)


def _mm2_body(aa_ref, ab_ref, z1_ref, d0_ref, d1_ref, wl_ref, wr_ref, bl_ref,
              y2_ref, z2_ref):
    dg = jnp.maximum(d0_ref[...][:, 0:1] + d1_ref[...][:, 0:1], 1.0)
    agg = jnp.concatenate([aa_ref[...], ab_ref[...]], axis=1)
    h1 = jnp.maximum(agg / dg + z1_ref[...], 0.0)
    y2_ref[...] = _dotT(h1, wl_ref[...])
    z2_ref[...] = _dotT(h1, wr_ref[...]) + bl_ref[...]


def _run_mm2(agg, z1, degw, wl2, wr2, bl2r):
    return pl.pallas_call(
        _mm2_body,
        grid=(_NB,),
        in_specs=[
            pl.BlockSpec((256, 128), lambda i: (i, 0)),
            pl.BlockSpec((256, 128), lambda i: (i + _NB, 0)),
            pl.BlockSpec((256, 256), lambda i: (i, 0)),
            pl.BlockSpec((256, 128), lambda i: (i, 0)),
            pl.BlockSpec((256, 128), lambda i: (i + _NB, 0)),
            pl.BlockSpec((128, 256), lambda i: (0, 0)),
            pl.BlockSpec((128, 256), lambda i: (0, 0)),
            pl.BlockSpec((1, 128), lambda i: (0, 0)),
        ],
        out_specs=[
            pl.BlockSpec((256, 128), lambda i: (i, 0)),
            pl.BlockSpec((256, 128), lambda i: (i, 0)),
        ],
        out_shape=[
            jax.ShapeDtypeStruct((_NP, 128), F32),
            jax.ShapeDtypeStruct((_NP, 128), F32),
        ],
    )(agg, agg, z1, degw, degw, wl2, wr2, bl2r)


def _head_body(p0_ref, p1_ref, z2_ref, d0_ref, d1_ref, bat_ref,
               w3_ref, b3_ref, w4_ref, b4_ref,
               out_ref, acc_ref, cnt_ref):
    i = pl.program_id(0)
    nb = pl.num_programs(0)

    @pl.when(i == 0)
    def _():
        acc_ref[...] = jnp.zeros_like(acc_ref)
        cnt_ref[...] = jnp.zeros_like(cnt_ref)

    dg = jnp.maximum(d0_ref[...][:, 0:1] + d1_ref[...][:, 0:1], 1.0)
    h2 = jnp.maximum((p0_ref[...] + p1_ref[...]) / dg + z2_ref[...], 0.0)
    b = bat_ref[0, 0, :]
    onehot = (b[:, None] == lax.broadcasted_iota(I32, (256, _G), 1)).astype(F32)
    acc_ref[...] += lax.dot_general(onehot, h2, (((0,), (0,)), ((), ())),
                                    preferred_element_type=F32)
    cnt_ref[...] += lax.dot_general(onehot, jnp.ones((256, 8), F32),
                                    (((0,), (0,)), ((), ())),
                                    preferred_element_type=F32)

    @pl.when(i == nb - 1)
    def _():
        pool = acc_ref[...] / jnp.maximum(cnt_ref[...][:, 0:1], 1.0)
        t3 = _dotT(pool, w3_ref[...]) + b3_ref[...]
        t4 = _dotT(t3, w4_ref[...]) + b4_ref[...]
        m = jnp.max(t4, axis=-1, keepdims=True)
        e = jnp.exp(t4 - m)
        out_ref[...] = e / jnp.sum(e, axis=-1, keepdims=True)


def _run_head(parts, z2, degw, bat3, w3, b3r, w4, b4r):
    return pl.pallas_call(
        _head_body,
        grid=(_NB,),
        in_specs=[
            pl.BlockSpec((256, 128), lambda i: (i, 0)),
            pl.BlockSpec((256, 128), lambda i: (i + _NB, 0)),
            pl.BlockSpec((256, 128), lambda i: (i, 0)),
            pl.BlockSpec((256, 128), lambda i: (i, 0)),
            pl.BlockSpec((256, 128), lambda i: (i + _NB, 0)),
            pl.BlockSpec((1, 1, 256), lambda i: (i, 0, 0)),
            pl.BlockSpec((64, 128), lambda i: (0, 0)),
            pl.BlockSpec((1, 64), lambda i: (0, 0)),
            pl.BlockSpec((2, 64), lambda i: (0, 0)),
            pl.BlockSpec((1, 2), lambda i: (0, 0)),
        ],
        out_specs=pl.BlockSpec((_G, 2), lambda i: (0, 0)),
        out_shape=jax.ShapeDtypeStruct((_G, 2), F32),
        scratch_shapes=[
            pltpu.VMEM((_G, 128), F32),
            pltpu.VMEM((_G, 8), F32),
        ],
    )(parts, parts, z2, degw, degw, bat3, w3, b3r, w4, b4r)


_MESH = plsc.VectorSubcoreMesh(core_axis_name="c", subcore_axis_name="s")


def _deg_body(dst_hbm, zf_hbm, onesf_hbm,
              degw_hbm,
              dst_v, ones_v, table, sem):
    c = lax.axis_index("c")
    s = lax.axis_index("s")
    f = c * _NS + s
    base = s * _ROWS_PER_TILE

    pltpu.sync_copy(zf_hbm, table.at[pl.ds(base, _ROWS_PER_TILE)])
    pltpu.sync_copy(onesf_hbm, ones_v)

    plsc.subcore_barrier()

    def outer(g, carry):
        pltpu.sync_copy(dst_hbm.at[pl.ds(f * _CH2 + g * _CHB2, _CHB2)], dst_v)
        for j in range(_CHB2):
            pltpu.sync_copy(ones_v, table.at[dst_v.at[j]], add=True)
        return carry

    lax.fori_loop(0, _CH2 // _CHB2, outer, 0)

    plsc.subcore_barrier()

    pltpu.sync_copy(table.at[pl.ds(base, _ROWS_PER_TILE)],
                    degw_hbm.at[pl.ds(c * _NP + base, _ROWS_PER_TILE)])


def _run_deg(dst2, zf, onesf):
    return pl.kernel(
        _deg_body,
        out_type=[jax.ShapeDtypeStruct((2 * _NP, 128), F32)],
        mesh=_MESH,
        scratch_types=[
            pltpu.VMEM((_CHB2, _K), I32),
            pltpu.VMEM((_K, 128), F32),
            pltpu.VMEM_SHARED((_NP, 128), F32),
            pltpu.SemaphoreType.DMA,
        ],
    )(dst2, zf, onesf)


def _pipelined_edges(table_hbm, src_hbm, dst_hbm, src_v, dst_v,
                     buf0, buf1, sem0, sem1, table,
                     idx_src0, idx_dst0, n_refill, chb):
    bufs = (buf0, buf1)
    sems = (sem0, sem1)

    def outer(g, carry):
        pltpu.sync_copy(src_hbm.at[pl.ds(idx_src0 + g * chb, chb)], src_v)
        pltpu.sync_copy(dst_hbm.at[pl.ds(idx_dst0 + g * chb, chb)], dst_v)
        cps = [None, None]
        cps[0] = pltpu.async_copy(table_hbm.at[src_v.at[0]], bufs[0], sems[0])
        cps[1] = pltpu.async_copy(table_hbm.at[src_v.at[1]], bufs[1], sems[1])
        for j in range(chb):
            cps[j % 2].wait()
            pltpu.sync_copy(bufs[j % 2], table.at[dst_v.at[j]], add=True)
            if j + 2 < chb:
                cps[j % 2] = pltpu.async_copy(
                    table_hbm.at[src_v.at[j + 2]],
                    bufs[j % 2], sems[j % 2])
        return carry

    lax.fori_loop(0, n_refill, outer, 0)


def _agg1_body(yall_hbm, src_hbm, dst_hbm, zf_hbm,
               agg_hbm,
               src_v, dst_v, buf0, buf1, table, sem0, sem1):
    c = lax.axis_index("c")
    s = lax.axis_index("s")
    base = s * _ROWS_PER_TILE

    pltpu.sync_copy(zf_hbm, table.at[pl.ds(base, _ROWS_PER_TILE)])

    plsc.subcore_barrier()

    _pipelined_edges(yall_hbm, src_hbm, dst_hbm, src_v, dst_v,
                     buf0, buf1, sem0, sem1, table,
                     c * (_NS * _CH1) + s * _CH1, s * _CH1,
                     _CH1 // _CHB1, _CHB1)

    plsc.subcore_barrier()

    pltpu.sync_copy(table.at[pl.ds(base, _ROWS_PER_TILE)],
                    agg_hbm.at[pl.ds(c * _NP + base, _ROWS_PER_TILE)])


def _run_agg1(yall, src1s, dst1, zf):
    return pl.kernel(
        _agg1_body,
        out_type=[jax.ShapeDtypeStruct((2 * _NP, 128), F32)],
        mesh=_MESH,
        scratch_types=[
            pltpu.VMEM((_CHB1, _K), I32),
            pltpu.VMEM((_CHB1, _K), I32),
            pltpu.VMEM((_K, 128), F32),
            pltpu.VMEM((_K, 128), F32),
            pltpu.VMEM_SHARED((_NP, 128), F32),
            pltpu.SemaphoreType.DMA,
            pltpu.SemaphoreType.DMA,
        ],
    )(yall, src1s, dst1, zf)


def _agg2_body(y2_hbm, src_hbm, dst_hbm, zf_hbm,
               parts_hbm,
               src_v, dst_v, buf0, buf1, table, sem0, sem1):
    c = lax.axis_index("c")
    s = lax.axis_index("s")
    f = c * _NS + s
    base = s * _ROWS_PER_TILE

    pltpu.sync_copy(zf_hbm, table.at[pl.ds(base, _ROWS_PER_TILE)])

    plsc.subcore_barrier()

    _pipelined_edges(y2_hbm, src_hbm, dst_hbm, src_v, dst_v,
                     buf0, buf1, sem0, sem1, table,
                     c * (_NC * _NS * _CH2) + f * _CH2, f * _CH2,
                     _CH2 // _CHB2, _CHB2)

    plsc.subcore_barrier()

    pltpu.sync_copy(table.at[pl.ds(base, _ROWS_PER_TILE)],
                    parts_hbm.at[pl.ds(c * _NP + base, _ROWS_PER_TILE)])


def _run_agg2(y2, src2, dst2, zf):
    return pl.kernel(
        _agg2_body,
        out_type=[jax.ShapeDtypeStruct((2 * _NP, 128), F32)],
        mesh=_MESH,
        scratch_types=[
            pltpu.VMEM((_CHB2, _K), I32),
            pltpu.VMEM((_CHB2, _K), I32),
            pltpu.VMEM((_K, 128), F32),
            pltpu.VMEM((_K, 128), F32),
            pltpu.VMEM_SHARED((_NP, 128), F32),
            pltpu.SemaphoreType.DMA,
            pltpu.SemaphoreType.DMA,
        ],
    )(y2, src2, dst2, zf)


def _edge_layout(v, groups, per_group_pad, chunks):
    v = v.reshape(groups, -1)
    pad = per_group_pad - v.shape[1]
    v = jnp.concatenate(
        [v, jnp.full((groups, pad), _PAD_IDX, dtype=I32)], axis=1)
    return v.reshape(groups * chunks, _K)


@jax.jit
def kernel(x, edge_index, batch, Wl1, bl1, Wr1, Wl2, bl2, Wr2, W3, b3, W4, b4):
    src = edge_index[0]
    dst = edge_index[1]

    xp = jnp.pad(x, ((0, _NP - _N), (0, 0)))
    src1 = _edge_layout(src, _NS, _EPT1_PAD, _CH1)
    src1s = jnp.concatenate([src1, src1 + _NP], axis=0)
    dst1 = _edge_layout(dst, _NS, _EPT1_PAD, _CH1)
    src2 = _edge_layout(src, _NC * _NS, _EPG2_PAD, _CH2)
    src2s = jnp.concatenate([src2, src2 + _NP], axis=0)
    dst2 = _edge_layout(dst, _NC * _NS, _EPG2_PAD, _CH2)
    batp = jnp.concatenate(
        [batch, jnp.full((_NP - _N,), _G, dtype=batch.dtype)]
    ).reshape(_NB, 1, 256)

    zf = jnp.zeros((_ROWS_PER_TILE, 128), F32)
    onesf = jnp.ones((_K, 128), F32)

    degw = _run_deg(dst2, zf, onesf)[0]
    ya, yb, z1 = _run_mm1(xp, Wl1[:128], Wl1[128:], Wr1, bl1.reshape(1, 256))
    yall = jnp.concatenate([ya, yb], axis=0)

    agg = _run_agg1(yall, src1s, dst1, zf)[0]
    y2, z2 = _run_mm2(agg, z1, degw, Wl2, Wr2, bl2.reshape(1, 128))
    y2d = jnp.concatenate([y2, y2], axis=0)
    parts = _run_agg2(y2d, src2s, dst2, zf)[0]
    out = _run_head(parts, z2, degw, batp, W3, b3.reshape(1, 64),
                    W4, b4.reshape(1, 2))
    return out
